# baseline (speedup 1.0000x reference)
import jax
import jax.numpy as jnp
from jax.experimental import pallas as pl
from jax.experimental.pallas import tpu as pltpu

_BLK = 2048
_BCHUNK = 512
_LOG2E = 1.4426950408889634


def _ctrl_kernel(x_ref, xl_ref, Wh_ref, bh_ref, Wg_ref, bg_ref, Wr_ref,
                 br_ref, h_ref, rk_ref, gate_ref):
    x = x_ref[...]
    h = jnp.tanh(jnp.dot(x, Wh_ref[...], preferred_element_type=jnp.float32)
                 + bh_ref[...])
    h_ref[...] = h
    rk = jnp.dot(h, Wr_ref[...], preferred_element_type=jnp.float32) + br_ref[...]
    knorm = jnp.sqrt(jnp.sum(rk * rk, axis=1, keepdims=True)) + 1e-8
    rk_ref[...] = rk * (_LOG2E / knorm)
    gate_ref[...] = (jnp.dot(xl_ref[...], Wg_ref[...],
                             preferred_element_type=jnp.float32) + bg_ref[...])


def _flash_kernel(rk_ref, M_ref, r_ref, siml_ref, l_ref, acc_ref, lsum_ref):
    i = pl.program_id(1)
    nb = pl.num_programs(1)

    @pl.when(i == 0)
    def _init():
        acc_ref[...] = jnp.zeros_like(acc_ref)
        lsum_ref[...] = jnp.zeros_like(lsum_ref)

    Mb = M_ref[...]
    msq = jnp.sum(Mb * Mb, axis=1, keepdims=True)
    Mn = Mb / (jnp.sqrt(msq) + 1e-8)
    sim2 = jax.lax.dot_general(rk_ref[...], Mn, (((1,), (1,)), ((), ())),
                               preferred_element_type=jnp.float32)
    p = jnp.exp2(sim2)
    lsum_ref[...] += jnp.sum(p, axis=1, keepdims=True)
    acc_ref[...] += jnp.dot(p.astype(jnp.bfloat16), Mb.astype(jnp.bfloat16),
                            preferred_element_type=jnp.float32)
    siml_ref[0] = sim2[-1:, :]

    @pl.when(i == nb - 1)
    def _fin():
        l = lsum_ref[...]
        r_ref[...] = acc_ref[...] / l
        l_ref[...] = l


def _final_kernel(h_ref, r_ref, Wo_ref, bo_ref, siml_ref, llast_ref,
                  out_ref, w_ref):
    cd = h_ref.shape[1]
    Wo = Wo_ref[...]
    out = (jnp.dot(h_ref[...], Wo[:cd, :], preferred_element_type=jnp.float32)
           + jnp.dot(r_ref[...], Wo[cd:, :], preferred_element_type=jnp.float32)
           + bo_ref[...])
    out_ref[...] = out
    w_ref[...] = jnp.exp2(siml_ref[...]) / llast_ref[...]


def kernel(x, W_h, b_h, W_g, b_g, W_r, b_r, M, W_o, b_o):
    B, _ = x.shape
    CD = W_h.shape[1]
    L, LS = M.shape
    nb = L // _BLK
    nbc = B // _BCHUNK

    bh2 = b_h.reshape(1, CD)
    bg2 = b_g.reshape(1, 1)
    br2 = b_r.reshape(1, LS)
    bo2 = b_o.reshape(1, 1)
    x_last = x[B - 1:B, :]

    h, rk, gate = pl.pallas_call(
        _ctrl_kernel,
        out_shape=(
            jax.ShapeDtypeStruct((B, CD), jnp.float32),
            jax.ShapeDtypeStruct((B, LS), jnp.float32),
            jax.ShapeDtypeStruct((1, 1), jnp.float32),
        ),
    )(x, x_last, W_h, bh2, W_g, bg2, W_r, br2)

    r, siml, l = pl.pallas_call(
        _flash_kernel,
        grid=(nbc, nb),
        in_specs=[
            pl.BlockSpec((_BCHUNK, LS), lambda b, i: (b, 0)),
            pl.BlockSpec((_BLK, LS), lambda b, i: (i, 0)),
        ],
        out_specs=[
            pl.BlockSpec((_BCHUNK, LS), lambda b, i: (b, 0)),
            pl.BlockSpec((1, 1, _BLK), lambda b, i: (b, 0, i)),
            pl.BlockSpec((_BCHUNK, 1), lambda b, i: (b, 0)),
        ],
        out_shape=(
            jax.ShapeDtypeStruct((B, LS), jnp.float32),
            jax.ShapeDtypeStruct((nbc, 1, L), jnp.float32),
            jax.ShapeDtypeStruct((B, 1), jnp.float32),
        ),
        scratch_shapes=[
            pltpu.VMEM((_BCHUNK, LS), jnp.float32),
            pltpu.VMEM((_BCHUNK, 1), jnp.float32),
        ],
        compiler_params=pltpu.CompilerParams(
            dimension_semantics=("parallel", "arbitrary"),
        ),
    )(rk, M)

    out, w = pl.pallas_call(
        _final_kernel,
        out_shape=(
            jax.ShapeDtypeStruct((B, 1), jnp.float32),
            jax.ShapeDtypeStruct((1, L), jnp.float32),
        ),
    )(h, r, W_o, bo2, siml[nbc - 1], l[B - 1:B, :])

    return (out[:, 0], h[B - 1], gate[0], w[0])


# speedup vs baseline: 1.1993x; 1.1993x over previous
import jax
import jax.numpy as jnp
from jax.experimental import pallas as pl
from jax.experimental.pallas import tpu as pltpu

_BLK = 2048
_LOG2E = 1.4426950408889634


def _ctrl_kernel(x_ref, xl_ref, Wh_ref, bh_ref, Wg_ref, bg_ref, Wr_ref,
                 br_ref, h_ref, rk_ref, gate_ref):
    x = x_ref[...]
    h = jnp.tanh(jnp.dot(x, Wh_ref[...], preferred_element_type=jnp.float32)
                 + bh_ref[...])
    h_ref[...] = h
    rk = jnp.dot(h, Wr_ref[...], preferred_element_type=jnp.float32) + br_ref[...]
    knorm = jnp.sqrt(jnp.sum(rk * rk, axis=1, keepdims=True)) + 1e-8
    rk_ref[...] = rk * (_LOG2E / knorm)
    gate_ref[...] = (jnp.dot(xl_ref[...], Wg_ref[...],
                             preferred_element_type=jnp.float32) + bg_ref[...])


def _flash_kernel(rk_ref, M_ref, r_ref, siml_ref, l_ref, acc_ref, lsum_ref):
    i = pl.program_id(0)
    nb = pl.num_programs(0)

    @pl.when(i == 0)
    def _init():
        acc_ref[...] = jnp.zeros_like(acc_ref)
        lsum_ref[...] = jnp.zeros_like(lsum_ref)

    Mb = M_ref[...]
    msq = jnp.sum(Mb * Mb, axis=1, keepdims=True)
    Mn = Mb / (jnp.sqrt(msq) + 1e-8)
    sim2 = jax.lax.dot_general(rk_ref[...], Mn, (((1,), (1,)), ((), ())),
                               preferred_element_type=jnp.float32)
    p = jnp.exp2(sim2)
    lsum_ref[...] += jnp.sum(p, axis=1, keepdims=True)
    acc_ref[...] += jnp.dot(p.astype(jnp.bfloat16), Mb.astype(jnp.bfloat16),
                            preferred_element_type=jnp.float32)
    siml_ref[...] = sim2[-1:, :]

    @pl.when(i == nb - 1)
    def _fin():
        l = lsum_ref[...]
        r_ref[...] = acc_ref[...] / l
        l_ref[...] = l


def _final_kernel(h_ref, r_ref, Wo_ref, bo_ref, siml_ref, llast_ref,
                  out_ref, w_ref):
    cd = h_ref.shape[1]
    Wo = Wo_ref[...]
    out = (jnp.dot(h_ref[...], Wo[:cd, :], preferred_element_type=jnp.float32)
           + jnp.dot(r_ref[...], Wo[cd:, :], preferred_element_type=jnp.float32)
           + bo_ref[...])
    out_ref[...] = out
    w_ref[...] = jnp.exp2(siml_ref[...]) / llast_ref[...]


def kernel(x, W_h, b_h, W_g, b_g, W_r, b_r, M, W_o, b_o):
    B, _ = x.shape
    CD = W_h.shape[1]
    L, LS = M.shape
    nb = L // _BLK

    bh2 = b_h.reshape(1, CD)
    bg2 = b_g.reshape(1, 1)
    br2 = b_r.reshape(1, LS)
    bo2 = b_o.reshape(1, 1)
    x_last = x[B - 1:B, :]

    h, rk, gate = pl.pallas_call(
        _ctrl_kernel,
        out_shape=(
            jax.ShapeDtypeStruct((B, CD), jnp.float32),
            jax.ShapeDtypeStruct((B, LS), jnp.float32),
            jax.ShapeDtypeStruct((1, 1), jnp.float32),
        ),
    )(x, x_last, W_h, bh2, W_g, bg2, W_r, br2)

    r, siml, l = pl.pallas_call(
        _flash_kernel,
        grid=(nb,),
        in_specs=[
            pl.BlockSpec((B, LS), lambda i: (0, 0)),
            pl.BlockSpec((_BLK, LS), lambda i: (i, 0)),
        ],
        out_specs=[
            pl.BlockSpec((B, LS), lambda i: (0, 0)),
            pl.BlockSpec((1, _BLK), lambda i: (0, i)),
            pl.BlockSpec((B, 1), lambda i: (0, 0)),
        ],
        out_shape=(
            jax.ShapeDtypeStruct((B, LS), jnp.float32),
            jax.ShapeDtypeStruct((1, L), jnp.float32),
            jax.ShapeDtypeStruct((B, 1), jnp.float32),
        ),
        scratch_shapes=[
            pltpu.VMEM((B, LS), jnp.float32),
            pltpu.VMEM((B, 1), jnp.float32),
        ],
    )(rk, M)

    out, w = pl.pallas_call(
        _final_kernel,
        out_shape=(
            jax.ShapeDtypeStruct((B, 1), jnp.float32),
            jax.ShapeDtypeStruct((1, L), jnp.float32),
        ),
    )(h, r, W_o, bo2, siml, l[B - 1:B, :])

    return (out[:, 0], h[B - 1], gate[0], w[0])


# speedup vs baseline: 1.2879x; 1.0738x over previous
import jax
import jax.numpy as jnp
from jax.experimental import pallas as pl
from jax.experimental.pallas import tpu as pltpu

_BLK = 4096
_LOG2E = 1.4426950408889634


def _ctrl_kernel(x_ref, xl_ref, Wh_ref, bh_ref, Wg_ref, bg_ref, Wr_ref,
                 br_ref, h_ref, rk_ref, gate_ref):
    x = x_ref[...]
    h = jnp.tanh(jnp.dot(x, Wh_ref[...], preferred_element_type=jnp.float32)
                 + bh_ref[...])
    h_ref[...] = h
    rk = jnp.dot(h, Wr_ref[...], preferred_element_type=jnp.float32) + br_ref[...]
    knorm = jnp.sqrt(jnp.sum(rk * rk, axis=1, keepdims=True)) + 1e-8
    rk_ref[...] = rk * (_LOG2E / knorm)
    gate_ref[...] = (jnp.dot(xl_ref[...], Wg_ref[...],
                             preferred_element_type=jnp.float32) + bg_ref[...])


def _flash_kernel(rk_ref, M_ref, r_ref, siml_ref, l_ref, acc_ref, lsum_ref):
    i = pl.program_id(0)
    nb = pl.num_programs(0)

    @pl.when(i == 0)
    def _init():
        acc_ref[...] = jnp.zeros_like(acc_ref)
        lsum_ref[...] = jnp.zeros_like(lsum_ref)

    Mb = M_ref[...]
    msq = jnp.sum(Mb * Mb, axis=1, keepdims=True)
    Mn = Mb * jax.lax.rsqrt(msq + 1e-16)
    sim2 = jax.lax.dot_general(rk_ref[...], Mn, (((1,), (1,)), ((), ())),
                               preferred_element_type=jnp.float32)
    p = jnp.exp2(sim2)
    lsum_ref[...] += jnp.sum(p, axis=1, keepdims=True)
    acc_ref[...] += jnp.dot(p.astype(jnp.bfloat16), Mb.astype(jnp.bfloat16),
                            preferred_element_type=jnp.float32)
    siml_ref[...] = sim2[-1:, :]

    @pl.when(i == nb - 1)
    def _fin():
        l = lsum_ref[...]
        r_ref[...] = acc_ref[...] / l
        l_ref[...] = l


def _final_kernel(h_ref, r_ref, Wo_ref, bo_ref, siml_ref, llast_ref,
                  out_ref, w_ref):
    cd = h_ref.shape[1]
    Wo = Wo_ref[...]
    out = (jnp.dot(h_ref[...], Wo[:cd, :], preferred_element_type=jnp.float32)
           + jnp.dot(r_ref[...], Wo[cd:, :], preferred_element_type=jnp.float32)
           + bo_ref[...])
    out_ref[...] = out
    w_ref[...] = jnp.exp2(siml_ref[...]) / llast_ref[...]


def kernel(x, W_h, b_h, W_g, b_g, W_r, b_r, M, W_o, b_o):
    B, _ = x.shape
    CD = W_h.shape[1]
    L, LS = M.shape
    nb = L // _BLK

    bh2 = b_h.reshape(1, CD)
    bg2 = b_g.reshape(1, 1)
    br2 = b_r.reshape(1, LS)
    bo2 = b_o.reshape(1, 1)
    x_last = x[B - 1:B, :]

    h, rk, gate = pl.pallas_call(
        _ctrl_kernel,
        out_shape=(
            jax.ShapeDtypeStruct((B, CD), jnp.float32),
            jax.ShapeDtypeStruct((B, LS), jnp.float32),
            jax.ShapeDtypeStruct((1, 1), jnp.float32),
        ),
    )(x, x_last, W_h, bh2, W_g, bg2, W_r, br2)

    r, siml, l = pl.pallas_call(
        _flash_kernel,
        grid=(nb,),
        in_specs=[
            pl.BlockSpec((B, LS), lambda i: (0, 0)),
            pl.BlockSpec((_BLK, LS), lambda i: (i, 0)),
        ],
        out_specs=[
            pl.BlockSpec((B, LS), lambda i: (0, 0)),
            pl.BlockSpec((1, _BLK), lambda i: (0, i)),
            pl.BlockSpec((B, 1), lambda i: (0, 0)),
        ],
        out_shape=(
            jax.ShapeDtypeStruct((B, LS), jnp.float32),
            jax.ShapeDtypeStruct((1, L), jnp.float32),
            jax.ShapeDtypeStruct((B, 1), jnp.float32),
        ),
        scratch_shapes=[
            pltpu.VMEM((B, LS), jnp.float32),
            pltpu.VMEM((B, 1), jnp.float32),
        ],
    )(rk, M)

    out, w = pl.pallas_call(
        _final_kernel,
        out_shape=(
            jax.ShapeDtypeStruct((B, 1), jnp.float32),
            jax.ShapeDtypeStruct((1, L), jnp.float32),
        ),
    )(h, r, W_o, bo2, siml, l[B - 1:B, :])

    return (out[:, 0], h[B - 1], gate[0], w[0])


# speedup vs baseline: 1.3955x; 1.0836x over previous
import jax
import jax.numpy as jnp
from jax.experimental import pallas as pl
from jax.experimental.pallas import tpu as pltpu

_BLK = 4096
_LOG2E = 1.4426950408889634


def _mann_kernel(x_ref, Wh_ref, bh_ref, Wg_ref, bg_ref, Wr_ref, br_ref,
                 M_ref, Wo_ref, bo_ref,
                 out_ref, hl_ref, gate_ref, w_ref,
                 h_ref, rk_ref, acc_ref, lsum_ref, siml_ref):
    i = pl.program_id(0)
    nb = pl.num_programs(0)
    B = x_ref.shape[0]

    @pl.when(i == 0)
    def _prologue():
        x = x_ref[...]
        h = jnp.tanh(jnp.dot(x, Wh_ref[...],
                             preferred_element_type=jnp.float32) + bh_ref[...])
        h_ref[...] = h
        rk = (jnp.dot(h, Wr_ref[...], preferred_element_type=jnp.float32)
              + br_ref[...])
        knorm = jnp.sqrt(jnp.sum(rk * rk, axis=1, keepdims=True)) + 1e-8
        rk_ref[...] = rk * (_LOG2E / knorm)
        gate_ref[...] = (jnp.dot(x[B - 1:B, :], Wg_ref[...],
                                 preferred_element_type=jnp.float32)
                         + bg_ref[...])
        acc_ref[...] = jnp.zeros_like(acc_ref)
        lsum_ref[...] = jnp.zeros_like(lsum_ref)

    Mb = M_ref[...]
    msq = jnp.sum(Mb * Mb, axis=1, keepdims=True)
    Mn = Mb * jax.lax.rsqrt(msq + 1e-16)
    sim2 = jax.lax.dot_general(rk_ref[...], Mn, (((1,), (1,)), ((), ())),
                               preferred_element_type=jnp.float32)
    p = jnp.exp2(sim2)
    lsum_ref[...] += jnp.sum(p, axis=1, keepdims=True)
    acc_ref[...] += jnp.dot(p.astype(jnp.bfloat16), Mb.astype(jnp.bfloat16),
                            preferred_element_type=jnp.float32)
    siml_ref[:, pl.ds(i * _BLK, _BLK)] = sim2[B - 1:B, :]

    @pl.when(i == nb - 1)
    def _epilogue():
        l = lsum_ref[...]
        r = acc_ref[...] / l
        h = h_ref[...]
        cd = h_ref.shape[1]
        Wo = Wo_ref[...]
        out_ref[...] = (jnp.dot(h, Wo[:cd, :],
                                preferred_element_type=jnp.float32)
                        + jnp.dot(r, Wo[cd:, :],
                                  preferred_element_type=jnp.float32)
                        + bo_ref[...])
        hl_ref[...] = h[B - 1:B, :]
        w_ref[...] = jnp.exp2(siml_ref[...]) / l[B - 1:B, :]


def kernel(x, W_h, b_h, W_g, b_g, W_r, b_r, M, W_o, b_o):
    B, _ = x.shape
    CD = W_h.shape[1]
    L, LS = M.shape
    nb = L // _BLK

    bh2 = b_h.reshape(1, CD)
    bg2 = b_g.reshape(1, 1)
    br2 = b_r.reshape(1, LS)
    bo2 = b_o.reshape(1, 1)

    const = lambda i: (0, 0)
    out, hl, gate, w = pl.pallas_call(
        _mann_kernel,
        grid=(nb,),
        in_specs=[
            pl.BlockSpec(x.shape, const),
            pl.BlockSpec(W_h.shape, const),
            pl.BlockSpec((1, CD), const),
            pl.BlockSpec(W_g.shape, const),
            pl.BlockSpec((1, 1), const),
            pl.BlockSpec(W_r.shape, const),
            pl.BlockSpec((1, LS), const),
            pl.BlockSpec((_BLK, LS), lambda i: (i, 0)),
            pl.BlockSpec(W_o.shape, const),
            pl.BlockSpec((1, 1), const),
        ],
        out_specs=[
            pl.BlockSpec((B, 1), const),
            pl.BlockSpec((1, CD), const),
            pl.BlockSpec((1, 1), const),
            pl.BlockSpec((1, L), const),
        ],
        out_shape=(
            jax.ShapeDtypeStruct((B, 1), jnp.float32),
            jax.ShapeDtypeStruct((1, CD), jnp.float32),
            jax.ShapeDtypeStruct((1, 1), jnp.float32),
            jax.ShapeDtypeStruct((1, L), jnp.float32),
        ),
        scratch_shapes=[
            pltpu.VMEM((B, CD), jnp.float32),
            pltpu.VMEM((B, LS), jnp.float32),
            pltpu.VMEM((B, LS), jnp.float32),
            pltpu.VMEM((B, 1), jnp.float32),
            pltpu.VMEM((1, L), jnp.float32),
        ],
    )(x, W_h, bh2, W_g, bg2, W_r, br2, M, W_o, bo2)

    return (out[:, 0], hl[0], gate[0], w[0])
